# initial kernel scaffold (unmeasured)
import functools

import jax
import jax.numpy as jnp
from jax import lax
from jax.experimental import pallas as pl
from jax.experimental.pallas import tpu as pltpu

N_DEV = 8
HEADS = 8
DH = 128
SQ = 2048
DM = 1024
QC = 512
BLK = 64
SCALE = 0.08838834764831843
CHUNK = SQ // N_DEV


def _body(x_ref, wq_ref, k_ref, v_ref, wo_ref, out_ref,
          comm, rs_send, rs_recv, ag_send, ag_recv):
    my = lax.axis_index("i")
    left = lax.rem(my + N_DEV - 1, N_DEV)
    right = lax.rem(my + 1, N_DEV)

    barrier = pltpu.get_barrier_semaphore()
    for nbr in (left, right):
        pl.semaphore_signal(barrier, inc=1, device_id=(nbr,),
                            device_id_type=pl.DeviceIdType.MESH)
    pl.semaphore_wait(barrier, 2)

    for qc in range(SQ // QC):
        rows = pl.ds(qc * QC, QC)
        kv_len = (qc + 1) * QC
        out_ref[rows, :] = jnp.zeros((QC, DM), jnp.float32)

        def head_body(h, _, qc=qc, rows=rows, kv_len=kv_len):
            q = jnp.dot(x_ref[rows, :], wq_ref[h],
                        preferred_element_type=jnp.float32)
            k = k_ref[h, pl.ds(0, kv_len), :]
            s = lax.dot_general(q, k, (((1,), (1,)), ((), ())),
                                preferred_element_type=jnp.float32) * SCALE
            rowi = qc * QC + lax.broadcasted_iota(jnp.int32, (QC, kv_len), 0)
            coli = lax.broadcasted_iota(jnp.int32, (QC, kv_len), 1)
            s = jnp.where((coli // BLK) <= (rowi // BLK), s, -1e9)
            m = jnp.max(s, axis=1, keepdims=True)
            p = jnp.exp(s - m)
            l = jnp.sum(p, axis=1, keepdims=True)
            ctx = jnp.dot(p, v_ref[h, pl.ds(0, kv_len), :],
                          preferred_element_type=jnp.float32) / l
            out_ref[rows, :] = out_ref[rows, :] + jnp.dot(
                ctx, wo_ref[h], preferred_element_type=jnp.float32)
            return 0

        lax.fori_loop(0, HEADS, head_body, 0)

    comm[0, :, :] = out_ref[pl.ds(my * CHUNK, CHUNK), :]
    for s in range(N_DEV - 1):
        rdma = pltpu.make_async_remote_copy(
            src_ref=comm.at[s],
            dst_ref=comm.at[s + 1],
            send_sem=rs_send.at[s],
            recv_sem=rs_recv.at[s],
            device_id=(right,),
            device_id_type=pl.DeviceIdType.MESH,
        )
        rdma.start()
        rdma.wait()
        c = lax.rem(my + 2 * N_DEV - 1 - s, N_DEV)
        comm[s + 1, :, :] = comm[s + 1, :, :] + out_ref[pl.ds(c * CHUNK, CHUNK), :]

    rc = lax.rem(my + 1, N_DEV)
    out_ref[pl.ds(rc * CHUNK, CHUNK), :] = comm[N_DEV - 1, :, :]

    for h in range(N_DEV - 1):
        send_slot = (N_DEV - 1 + h) % N_DEV
        recv_slot = h
        rdma = pltpu.make_async_remote_copy(
            src_ref=comm.at[send_slot],
            dst_ref=comm.at[recv_slot],
            send_sem=ag_send.at[h],
            recv_sem=ag_recv.at[h],
            device_id=(right,),
            device_id_type=pl.DeviceIdType.MESH,
        )
        rdma.start()
        rdma.wait()
        c = lax.rem(my + 2 * N_DEV - h, N_DEV)
        out_ref[pl.ds(c * CHUNK, CHUNK), :] = comm[recv_slot, :, :]

    @functools.partial(pl.run_scoped, sem=pltpu.SemaphoreType.REGULAR)
    def _(sem):
        for nbr in (left, right):
            pl.semaphore_signal(sem, inc=1, device_id=(nbr,),
                                device_id_type=pl.DeviceIdType.MESH)
        pl.semaphore_wait(sem, 2)


def kernel(x, Wq, K_ext, V_ext, Wo):
    i = lax.axis_index("i")
    x2 = x[0]
    k_loc = lax.dynamic_slice_in_dim(K_ext[0], i * HEADS, HEADS, axis=1)
    v_loc = lax.dynamic_slice_in_dim(V_ext[0], i * HEADS, HEADS, axis=1)
    k_loc = jnp.transpose(k_loc, (1, 0, 2))
    v_loc = jnp.transpose(v_loc, (1, 0, 2))
    wq_h = jnp.transpose(Wq.reshape(DM, HEADS, DH), (1, 0, 2))
    wo_h = Wo.reshape(HEADS, DH, DM)

    out = pl.pallas_call(
        _body,
        out_shape=jax.ShapeDtypeStruct((SQ, DM), jnp.float32),
        in_specs=[pl.BlockSpec(memory_space=pltpu.VMEM)] * 5,
        out_specs=pl.BlockSpec(memory_space=pltpu.VMEM),
        scratch_shapes=[
            pltpu.VMEM((N_DEV, CHUNK, DM), jnp.float32),
            pltpu.SemaphoreType.DMA((N_DEV - 1,)),
            pltpu.SemaphoreType.DMA((N_DEV - 1,)),
            pltpu.SemaphoreType.DMA((N_DEV - 1,)),
            pltpu.SemaphoreType.DMA((N_DEV - 1,)),
        ],
        compiler_params=pltpu.CompilerParams(collective_id=0),
    )(x2, wq_h, k_loc, v_loc, wo_h)
    return out[None]


# baseline (device time: 340818 ns/iter reference)
import functools

import jax
import jax.numpy as jnp
from jax import lax
from jax.experimental import pallas as pl
from jax.experimental.pallas import tpu as pltpu

N_DEV = 8
HEADS = 8
DH = 128
SQ = 2048
DM = 1024
QC = 256
BLK = 64
SCALE = 0.08838834764831843
CHUNK = SQ // N_DEV


def _body(x_ref, wq_ref, k_ref, v_ref, wo_ref, out_ref,
          comm, rs_send, rs_recv, ag_send, ag_recv):
    my = lax.axis_index("i")
    left = lax.rem(my + N_DEV - 1, N_DEV)
    right = lax.rem(my + 1, N_DEV)

    barrier = pltpu.get_barrier_semaphore()
    for nbr in (left, right):
        pl.semaphore_signal(barrier, inc=1, device_id=(nbr,),
                            device_id_type=pl.DeviceIdType.MESH)
    pl.semaphore_wait(barrier, 2)

    for qc in range(SQ // QC):
        rows = pl.ds(qc * QC, QC)
        kv_len = (qc + 1) * QC
        out_ref[rows, :] = jnp.zeros((QC, DM), jnp.float32)

        def head_body(h, _, qc=qc, rows=rows, kv_len=kv_len):
            q = jnp.dot(x_ref[rows, :], wq_ref[h],
                        preferred_element_type=jnp.float32)
            k = k_ref[h, pl.ds(0, kv_len), :]
            s = lax.dot_general(q, k, (((1,), (1,)), ((), ())),
                                preferred_element_type=jnp.float32) * SCALE
            rowi = qc * QC + lax.broadcasted_iota(jnp.int32, (QC, kv_len), 0)
            coli = lax.broadcasted_iota(jnp.int32, (QC, kv_len), 1)
            s = jnp.where((coli // BLK) <= (rowi // BLK), s, -1e9)
            m = jnp.max(s, axis=1, keepdims=True)
            p = jnp.exp(s - m)
            l = jnp.sum(p, axis=1, keepdims=True)
            ctx = jnp.dot(p, v_ref[h, pl.ds(0, kv_len), :],
                          preferred_element_type=jnp.float32) / l
            out_ref[rows, :] = out_ref[rows, :] + jnp.dot(
                ctx, wo_ref[h], preferred_element_type=jnp.float32)
            return 0

        lax.fori_loop(0, HEADS, head_body, 0)

    comm[0, :, :] = out_ref[pl.ds(my * CHUNK, CHUNK), :]
    for s in range(N_DEV - 1):
        rdma = pltpu.make_async_remote_copy(
            src_ref=comm.at[s],
            dst_ref=comm.at[s + 1],
            send_sem=rs_send.at[s],
            recv_sem=rs_recv.at[s],
            device_id=(right,),
            device_id_type=pl.DeviceIdType.MESH,
        )
        rdma.start()
        rdma.wait()
        c = lax.rem(my + 2 * N_DEV - 1 - s, N_DEV)
        comm[s + 1, :, :] = comm[s + 1, :, :] + out_ref[pl.ds(c * CHUNK, CHUNK), :]

    rc = lax.rem(my + 1, N_DEV)
    out_ref[pl.ds(rc * CHUNK, CHUNK), :] = comm[N_DEV - 1, :, :]

    for h in range(N_DEV - 1):
        send_slot = (N_DEV - 1 + h) % N_DEV
        recv_slot = h
        rdma = pltpu.make_async_remote_copy(
            src_ref=comm.at[send_slot],
            dst_ref=comm.at[recv_slot],
            send_sem=ag_send.at[h],
            recv_sem=ag_recv.at[h],
            device_id=(right,),
            device_id_type=pl.DeviceIdType.MESH,
        )
        rdma.start()
        rdma.wait()
        c = lax.rem(my + 2 * N_DEV - h, N_DEV)
        out_ref[pl.ds(c * CHUNK, CHUNK), :] = comm[recv_slot, :, :]

    @functools.partial(pl.run_scoped, sem=pltpu.SemaphoreType.REGULAR)
    def _(sem):
        for nbr in (left, right):
            pl.semaphore_signal(sem, inc=1, device_id=(nbr,),
                                device_id_type=pl.DeviceIdType.MESH)
        pl.semaphore_wait(sem, 2)


def kernel(x, Wq, K_ext, V_ext, Wo):
    i = lax.axis_index("i")
    x2 = x[0]
    k_loc = lax.dynamic_slice_in_dim(K_ext[0], i * HEADS, HEADS, axis=1)
    v_loc = lax.dynamic_slice_in_dim(V_ext[0], i * HEADS, HEADS, axis=1)
    k_loc = jnp.transpose(k_loc, (1, 0, 2))
    v_loc = jnp.transpose(v_loc, (1, 0, 2))
    wq_h = jnp.transpose(Wq.reshape(DM, HEADS, DH), (1, 0, 2))
    wo_h = Wo.reshape(HEADS, DH, DM)

    out = pl.pallas_call(
        _body,
        out_shape=jax.ShapeDtypeStruct((SQ, DM), jnp.float32),
        in_specs=[pl.BlockSpec(memory_space=pltpu.VMEM)] * 5,
        out_specs=pl.BlockSpec(memory_space=pltpu.VMEM),
        scratch_shapes=[
            pltpu.VMEM((N_DEV, CHUNK, DM), jnp.float32),
            pltpu.SemaphoreType.DMA((N_DEV - 1,)),
            pltpu.SemaphoreType.DMA((N_DEV - 1,)),
            pltpu.SemaphoreType.DMA((N_DEV - 1,)),
            pltpu.SemaphoreType.DMA((N_DEV - 1,)),
        ],
        compiler_params=pltpu.CompilerParams(
            collective_id=0,
            vmem_limit_bytes=50 * 1024 * 1024,
        ),
    )(x2, wq_h, k_loc, v_loc, wo_h)
    return out[None]


# device time: 304987 ns/iter; 1.1175x vs baseline; 1.1175x over previous
import functools

import jax
import jax.numpy as jnp
from jax import lax
from jax.experimental import pallas as pl
from jax.experimental.pallas import tpu as pltpu

N_DEV = 8
HEADS = 8
DH = 128
SQ = 2048
DM = 1024
BLK = 64
SCALE = 0.08838834764831843
CHUNK = SQ // N_DEV
HALF = DM // 2


def _body(x_ref, wq_ref, k_ref, v_ref, wo_ref, out_ref,
          comm, agr, agl, rs_send, rs_recv,
          agr_send, agr_recv, agl_send, agl_recv):
    my = lax.axis_index("i")
    left = lax.rem(my + N_DEV - 1, N_DEV)
    right = lax.rem(my + 1, N_DEV)

    barrier = pltpu.get_barrier_semaphore()
    for nbr in (left, right):
        pl.semaphore_signal(barrier, inc=1, device_id=(nbr,),
                            device_id_type=pl.DeviceIdType.MESH)
    pl.semaphore_wait(barrier, 2)

    def compute_chunk(c):
        row0 = c * CHUNK
        xc = x_ref[pl.ds(row0, CHUNK), :]

        def head_body(h, acc):
            q = jnp.dot(xc, wq_ref[h], preferred_element_type=jnp.float32)

            def kv_body(kvb, carry):
                m, l, o = carry
                kv0 = kvb * CHUNK
                k = k_ref[h, pl.ds(kv0, CHUNK), :]
                s = lax.dot_general(q, k, (((1,), (1,)), ((), ())),
                                    preferred_element_type=jnp.float32) * SCALE
                rowi = row0 + lax.broadcasted_iota(jnp.int32, (CHUNK, CHUNK), 0)
                coli = kv0 + lax.broadcasted_iota(jnp.int32, (CHUNK, CHUNK), 1)
                s = jnp.where((coli // BLK) <= (rowi // BLK), s, -1e9)
                m_new = jnp.maximum(m, jnp.max(s, axis=1, keepdims=True))
                corr = jnp.exp(m - m_new)
                p = jnp.exp(s - m_new)
                l_new = l * corr + jnp.sum(p, axis=1, keepdims=True)
                o_new = o * corr + jnp.dot(
                    p, v_ref[h, pl.ds(kv0, CHUNK), :],
                    preferred_element_type=jnp.float32)
                return m_new, l_new, o_new

            m0 = jnp.full((CHUNK, 1), -1e30, jnp.float32)
            l0 = jnp.zeros((CHUNK, 1), jnp.float32)
            o0 = jnp.zeros((CHUNK, DH), jnp.float32)
            m, l, o = lax.fori_loop(0, c + 1, kv_body, (m0, l0, o0))
            ctx = o / l
            return acc + jnp.dot(ctx, wo_ref[h], preferred_element_type=jnp.float32)

        acc = lax.fori_loop(0, HEADS, head_body,
                            jnp.zeros((CHUNK, DM), jnp.float32))
        out_ref[pl.ds(row0, CHUNK), :] = acc

    compute_chunk(my)
    comm[0, :, :] = out_ref[pl.ds(my * CHUNK, CHUNK), :]

    def rs_body(s, _):
        rdma = pltpu.make_async_remote_copy(
            src_ref=comm.at[s],
            dst_ref=comm.at[s + 1],
            send_sem=rs_send.at[s],
            recv_sem=rs_recv.at[s],
            device_id=(left,),
            device_id_type=pl.DeviceIdType.MESH,
        )
        rdma.start()
        cc = lax.rem(my + s + 1, N_DEV)
        compute_chunk(cc)
        rdma.wait()
        comm[s + 1, :, :] = comm[s + 1, :, :] + out_ref[pl.ds(cc * CHUNK, CHUNK), :]
        return 0

    lax.fori_loop(0, N_DEV - 1, rs_body, 0)

    rc = left
    out_ref[pl.ds(rc * CHUNK, CHUNK), :] = comm[N_DEV - 1, :, :]

    agr[0, :, :] = comm[N_DEV - 1, :, 0:HALF]
    agl[0, :, :] = comm[N_DEV - 1, :, HALF:DM]

    def ag_body(h, _):
        r_rdma = pltpu.make_async_remote_copy(
            src_ref=agr.at[h],
            dst_ref=agr.at[h + 1],
            send_sem=agr_send.at[h],
            recv_sem=agr_recv.at[h],
            device_id=(right,),
            device_id_type=pl.DeviceIdType.MESH,
        )
        l_rdma = pltpu.make_async_remote_copy(
            src_ref=agl.at[h],
            dst_ref=agl.at[h + 1],
            send_sem=agl_send.at[h],
            recv_sem=agl_recv.at[h],
            device_id=(left,),
            device_id_type=pl.DeviceIdType.MESH,
        )
        r_rdma.start()
        l_rdma.start()
        r_rdma.wait()
        l_rdma.wait()
        cr = lax.rem(my + 2 * N_DEV - 2 - h, N_DEV)
        cl = lax.rem(my + h, N_DEV)
        out_ref[pl.ds(cr * CHUNK, CHUNK), 0:HALF] = agr[h + 1, :, :]
        out_ref[pl.ds(cl * CHUNK, CHUNK), HALF:DM] = agl[h + 1, :, :]
        return 0

    lax.fori_loop(0, N_DEV - 1, ag_body, 0)

    @functools.partial(pl.run_scoped, sem=pltpu.SemaphoreType.REGULAR)
    def _(sem):
        for nbr in (left, right):
            pl.semaphore_signal(sem, inc=1, device_id=(nbr,),
                                device_id_type=pl.DeviceIdType.MESH)
        pl.semaphore_wait(sem, 2)


def kernel(x, Wq, K_ext, V_ext, Wo):
    i = lax.axis_index("i")
    x2 = x[0]
    k_loc = lax.dynamic_slice_in_dim(K_ext[0], i * HEADS, HEADS, axis=1)
    v_loc = lax.dynamic_slice_in_dim(V_ext[0], i * HEADS, HEADS, axis=1)
    k_loc = jnp.transpose(k_loc, (1, 0, 2))
    v_loc = jnp.transpose(v_loc, (1, 0, 2))
    wq_h = jnp.transpose(Wq.reshape(DM, HEADS, DH), (1, 0, 2))
    wo_h = Wo.reshape(HEADS, DH, DM)

    out = pl.pallas_call(
        _body,
        out_shape=jax.ShapeDtypeStruct((SQ, DM), jnp.float32),
        in_specs=[pl.BlockSpec(memory_space=pltpu.VMEM)] * 5,
        out_specs=pl.BlockSpec(memory_space=pltpu.VMEM),
        scratch_shapes=[
            pltpu.VMEM((N_DEV, CHUNK, DM), jnp.float32),
            pltpu.VMEM((N_DEV, CHUNK, HALF), jnp.float32),
            pltpu.VMEM((N_DEV, CHUNK, HALF), jnp.float32),
            pltpu.SemaphoreType.DMA((N_DEV - 1,)),
            pltpu.SemaphoreType.DMA((N_DEV - 1,)),
            pltpu.SemaphoreType.DMA((N_DEV - 1,)),
            pltpu.SemaphoreType.DMA((N_DEV - 1,)),
            pltpu.SemaphoreType.DMA((N_DEV - 1,)),
            pltpu.SemaphoreType.DMA((N_DEV - 1,)),
        ],
        compiler_params=pltpu.CompilerParams(
            collective_id=0,
            vmem_limit_bytes=64 * 1024 * 1024,
        ),
    )(x2, wq_h, k_loc, v_loc, wo_h)
    return out[None]


# device time: 233584 ns/iter; 1.4591x vs baseline; 1.3057x over previous
import functools

import jax
import jax.numpy as jnp
from jax import lax
from jax.experimental import pallas as pl
from jax.experimental.pallas import tpu as pltpu

N_DEV = 8
HEADS = 8
DH = 128
SQ = 2048
DM = 1024
BLK = 64
SCALE = 0.08838834764831843
CHUNK = SQ // N_DEV
HALF = DM // 2


def _body(x_ref, wq_ref, k_ref, v_ref, wo_ref, out_ref,
          comm, agr, agl, rs_send, rs_recv,
          agr_send, agr_recv, agl_send, agl_recv):
    my = lax.axis_index("i")
    left = lax.rem(my + N_DEV - 1, N_DEV)
    right = lax.rem(my + 1, N_DEV)

    barrier = pltpu.get_barrier_semaphore()
    for nbr in (left, right):
        pl.semaphore_signal(barrier, inc=1, device_id=(nbr,),
                            device_id_type=pl.DeviceIdType.MESH)
    pl.semaphore_wait(barrier, 2)

    def compute_chunk(c):
        row0 = c * CHUNK
        xc = x_ref[pl.ds(row0, CHUNK), :]

        for nb in (1, 2, 3, 4):
            @pl.when(c // 2 + 1 == nb)
            def _(nb=nb):
                kv_len = nb * 512

                def head_body(h, acc):
                    q = jnp.dot(xc, wq_ref[h],
                                preferred_element_type=jnp.float32)
                    k = k_ref[h, 0:kv_len, :]
                    s = lax.dot_general(q, k, (((1,), (1,)), ((), ())),
                                        preferred_element_type=jnp.float32) * SCALE
                    rowi = row0 + lax.broadcasted_iota(
                        jnp.int32, (CHUNK, kv_len), 0)
                    coli = lax.broadcasted_iota(jnp.int32, (CHUNK, kv_len), 1)
                    s = jnp.where((coli // BLK) <= (rowi // BLK), s, -1e9)
                    m = jnp.max(s, axis=1, keepdims=True)
                    p = jnp.exp(s - m)
                    l = jnp.sum(p, axis=1, keepdims=True)
                    ctx = jnp.dot(p, v_ref[h, 0:kv_len, :],
                                  preferred_element_type=jnp.float32) / l
                    return acc + jnp.dot(ctx, wo_ref[h],
                                         preferred_element_type=jnp.float32)

                acc = lax.fori_loop(0, HEADS, head_body,
                                    jnp.zeros((CHUNK, DM), jnp.float32))
                out_ref[pl.ds(row0, CHUNK), :] = acc

    compute_chunk(my)
    comm[0, :, :] = out_ref[pl.ds(my * CHUNK, CHUNK), :]

    def rs_body(s, _):
        rdma = pltpu.make_async_remote_copy(
            src_ref=comm.at[s],
            dst_ref=comm.at[s + 1],
            send_sem=rs_send.at[s],
            recv_sem=rs_recv.at[s],
            device_id=(left,),
            device_id_type=pl.DeviceIdType.MESH,
        )
        rdma.start()
        cc = lax.rem(my + s + 1, N_DEV)
        compute_chunk(cc)
        rdma.wait()
        comm[s + 1, :, :] = comm[s + 1, :, :] + out_ref[pl.ds(cc * CHUNK, CHUNK), :]
        return 0

    lax.fori_loop(0, N_DEV - 1, rs_body, 0)

    rc = left
    out_ref[pl.ds(rc * CHUNK, CHUNK), :] = comm[N_DEV - 1, :, :]

    agr[0, :, :] = comm[N_DEV - 1, :, 0:HALF]
    agl[0, :, :] = comm[N_DEV - 1, :, HALF:DM]

    def ag_body(h, _):
        r_rdma = pltpu.make_async_remote_copy(
            src_ref=agr.at[h],
            dst_ref=agr.at[h + 1],
            send_sem=agr_send.at[h],
            recv_sem=agr_recv.at[h],
            device_id=(right,),
            device_id_type=pl.DeviceIdType.MESH,
        )
        l_rdma = pltpu.make_async_remote_copy(
            src_ref=agl.at[h],
            dst_ref=agl.at[h + 1],
            send_sem=agl_send.at[h],
            recv_sem=agl_recv.at[h],
            device_id=(left,),
            device_id_type=pl.DeviceIdType.MESH,
        )
        r_rdma.start()
        l_rdma.start()
        r_rdma.wait()
        l_rdma.wait()
        cr = lax.rem(my + 2 * N_DEV - 2 - h, N_DEV)
        cl = lax.rem(my + h, N_DEV)
        out_ref[pl.ds(cr * CHUNK, CHUNK), 0:HALF] = agr[h + 1, :, :]
        out_ref[pl.ds(cl * CHUNK, CHUNK), HALF:DM] = agl[h + 1, :, :]
        return 0

    lax.fori_loop(0, N_DEV - 1, ag_body, 0)

    @functools.partial(pl.run_scoped, sem=pltpu.SemaphoreType.REGULAR)
    def _(sem):
        for nbr in (left, right):
            pl.semaphore_signal(sem, inc=1, device_id=(nbr,),
                                device_id_type=pl.DeviceIdType.MESH)
        pl.semaphore_wait(sem, 2)


def kernel(x, Wq, K_ext, V_ext, Wo):
    i = lax.axis_index("i")
    x2 = x[0]
    k_loc = lax.dynamic_slice_in_dim(K_ext[0], i * HEADS, HEADS, axis=1)
    v_loc = lax.dynamic_slice_in_dim(V_ext[0], i * HEADS, HEADS, axis=1)
    k_loc = jnp.transpose(k_loc, (1, 0, 2))
    v_loc = jnp.transpose(v_loc, (1, 0, 2))
    wq_h = jnp.transpose(Wq.reshape(DM, HEADS, DH), (1, 0, 2))
    wo_h = Wo.reshape(HEADS, DH, DM)

    out = pl.pallas_call(
        _body,
        out_shape=jax.ShapeDtypeStruct((SQ, DM), jnp.float32),
        in_specs=[pl.BlockSpec(memory_space=pltpu.VMEM)] * 5,
        out_specs=pl.BlockSpec(memory_space=pltpu.VMEM),
        scratch_shapes=[
            pltpu.VMEM((N_DEV, CHUNK, DM), jnp.float32),
            pltpu.VMEM((N_DEV, CHUNK, HALF), jnp.float32),
            pltpu.VMEM((N_DEV, CHUNK, HALF), jnp.float32),
            pltpu.SemaphoreType.DMA((N_DEV - 1,)),
            pltpu.SemaphoreType.DMA((N_DEV - 1,)),
            pltpu.SemaphoreType.DMA((N_DEV - 1,)),
            pltpu.SemaphoreType.DMA((N_DEV - 1,)),
            pltpu.SemaphoreType.DMA((N_DEV - 1,)),
            pltpu.SemaphoreType.DMA((N_DEV - 1,)),
        ],
        compiler_params=pltpu.CompilerParams(
            collective_id=0,
            vmem_limit_bytes=64 * 1024 * 1024,
        ),
    )(x2, wq_h, k_loc, v_loc, wo_h)
    return out[None]


# device time: 227901 ns/iter; 1.4955x vs baseline; 1.0249x over previous
import functools

import jax
import jax.numpy as jnp
from jax import lax
from jax.experimental import pallas as pl
from jax.experimental.pallas import tpu as pltpu

N_DEV = 8
HEADS = 8
DH = 128
SQ = 2048
DM = 1024
BLK = 64
SCALE = 0.08838834764831843
CHUNK = SQ // N_DEV


def _body(x_ref, wq_ref, k_ref, v_ref, wo_ref, out_ref,
          comm, rs_send, rs_recv, ag_send, ag_recv):
    my = lax.axis_index("i")
    left = lax.rem(my + N_DEV - 1, N_DEV)
    right = lax.rem(my + 1, N_DEV)
    chord = lax.rem(my + 4, N_DEV)

    barrier = pltpu.get_barrier_semaphore()
    for nbr in (left, right, chord):
        pl.semaphore_signal(barrier, inc=1, device_id=(nbr,),
                            device_id_type=pl.DeviceIdType.MESH)
    pl.semaphore_wait(barrier, 3)

    def compute_chunk(c):
        row0 = c * CHUNK
        xc = x_ref[pl.ds(row0, CHUNK), :]
        q_all = jnp.dot(xc, wq_ref[:, :],
                        preferred_element_type=jnp.float32)

        for nb in (1, 2, 3, 4):
            @pl.when(c // 2 + 1 == nb)
            def _(nb=nb):
                kv_len = nb * 512
                rowi = row0 + lax.broadcasted_iota(jnp.int32, (CHUNK, kv_len), 0)
                coli = lax.broadcasted_iota(jnp.int32, (CHUNK, kv_len), 1)
                bias = jnp.where((coli // BLK) <= (rowi // BLK),
                                 0.0, -1e9).astype(jnp.float32)

                ctxs = []
                for h in range(HEADS):
                    q = q_all[:, h * DH:(h + 1) * DH]
                    k = k_ref[h, 0:kv_len, :]
                    s = lax.dot_general(q, k, (((1,), (1,)), ((), ())),
                                        preferred_element_type=jnp.float32)
                    s = s * SCALE + bias
                    m = jnp.max(s, axis=1, keepdims=True)
                    p = jnp.exp(s - m)
                    l = jnp.sum(p, axis=1, keepdims=True)
                    ctxs.append(jnp.dot(p, v_ref[h, 0:kv_len, :],
                                        preferred_element_type=jnp.float32) / l)

                ctx_all = jnp.concatenate(ctxs, axis=1)
                out_ref[pl.ds(row0, CHUNK), :] = jnp.dot(
                    ctx_all, wo_ref[:, :], preferred_element_type=jnp.float32)

    compute_chunk(my)
    comm[0, :, :] = out_ref[pl.ds(my * CHUNK, CHUNK), :]

    def rs_body(s, _):
        rdma = pltpu.make_async_remote_copy(
            src_ref=comm.at[s],
            dst_ref=comm.at[s + 1],
            send_sem=rs_send.at[s],
            recv_sem=rs_recv.at[s],
            device_id=(left,),
            device_id_type=pl.DeviceIdType.MESH,
        )
        rdma.start()
        cc = lax.rem(my + s + 1, N_DEV)
        compute_chunk(cc)
        rdma.wait()
        comm[s + 1, :, :] = comm[s + 1, :, :] + out_ref[pl.ds(cc * CHUNK, CHUNK), :]
        return 0

    lax.fori_loop(0, N_DEV - 1, rs_body, 0)

    rc = left
    out_ref[pl.ds(rc * CHUNK, CHUNK), :] = comm[N_DEV - 1, :, :]

    def ag_copy(chunk_id, dev, si):
        r0 = chunk_id * CHUNK
        return pltpu.make_async_remote_copy(
            src_ref=out_ref.at[pl.ds(r0, CHUNK), :],
            dst_ref=out_ref.at[pl.ds(r0, CHUNK), :],
            send_sem=ag_send.at[si],
            recv_sem=ag_recv.at[si],
            device_id=(dev,),
            device_id_type=pl.DeviceIdType.MESH,
        )

    a_r = ag_copy(rc, right, 0)
    a_l = ag_copy(rc, left, 1)
    a_c = ag_copy(rc, chord, 2)
    a_r.start()
    a_l.start()
    a_c.start()

    a_r.wait()
    b_fl = ag_copy(lax.rem(my + N_DEV - 2, N_DEV), right, 3)
    b_fl.start()
    a_c.wait()
    b_cr = ag_copy(lax.rem(my + 3, N_DEV), right, 5)
    b_cl = ag_copy(lax.rem(my + 3, N_DEV), left, 6)
    b_cr.start()
    b_cl.start()
    a_l.wait()
    b_fr = ag_copy(my, left, 4)
    b_fr.start()

    b_fl.wait()
    b_fr.wait()
    b_cr.wait()
    b_cl.wait()

    @functools.partial(pl.run_scoped, sem=pltpu.SemaphoreType.REGULAR)
    def _(sem):
        for nbr in (left, right, chord):
            pl.semaphore_signal(sem, inc=1, device_id=(nbr,),
                                device_id_type=pl.DeviceIdType.MESH)
        pl.semaphore_wait(sem, 3)


def kernel(x, Wq, K_ext, V_ext, Wo):
    i = lax.axis_index("i")
    x2 = x[0]
    k_loc = lax.dynamic_slice_in_dim(K_ext[0], i * HEADS, HEADS, axis=1)
    v_loc = lax.dynamic_slice_in_dim(V_ext[0], i * HEADS, HEADS, axis=1)
    k_loc = jnp.transpose(k_loc, (1, 0, 2))
    v_loc = jnp.transpose(v_loc, (1, 0, 2))

    out = pl.pallas_call(
        _body,
        out_shape=jax.ShapeDtypeStruct((SQ, DM), jnp.float32),
        in_specs=[pl.BlockSpec(memory_space=pltpu.VMEM)] * 5,
        out_specs=pl.BlockSpec(memory_space=pltpu.VMEM),
        scratch_shapes=[
            pltpu.VMEM((N_DEV, CHUNK, DM), jnp.float32),
            pltpu.SemaphoreType.DMA((N_DEV - 1,)),
            pltpu.SemaphoreType.DMA((N_DEV - 1,)),
            pltpu.SemaphoreType.DMA((N_DEV - 1,)),
            pltpu.SemaphoreType.DMA((N_DEV - 1,)),
        ],
        compiler_params=pltpu.CompilerParams(
            collective_id=0,
            vmem_limit_bytes=64 * 1024 * 1024,
        ),
    )(x2, Wq, k_loc, v_loc, Wo)
    return out[None]


# device time: 190854 ns/iter; 1.7858x vs baseline; 1.1941x over previous
import functools

import jax
import jax.numpy as jnp
from jax import lax
from jax.experimental import pallas as pl
from jax.experimental.pallas import tpu as pltpu

N_DEV = 8
HEADS = 8
DH = 128
SQ = 2048
DM = 1024
BLK = 64
SCALE = 0.08838834764831843
CHUNK = SQ // N_DEV
HALF = DM // 2


def _body(x_ref, wq_ref, k_ref, v_ref, wo_ref, out_ref,
          comm_l, comm_r,
          rsl_send, rsl_recv, rsr_send, rsr_recv,
          agr_send, agr_recv, agl_send, agl_recv):
    my = lax.axis_index("i")
    left = lax.rem(my + N_DEV - 1, N_DEV)
    right = lax.rem(my + 1, N_DEV)

    barrier = pltpu.get_barrier_semaphore()
    for nbr in (left, right):
        pl.semaphore_signal(barrier, inc=1, device_id=(nbr,),
                            device_id_type=pl.DeviceIdType.MESH)
    pl.semaphore_wait(barrier, 2)

    def compute_chunk(c):
        row0 = c * CHUNK
        xc = x_ref[pl.ds(row0, CHUNK), :]
        q_all = jnp.dot(xc, wq_ref[:, :],
                        preferred_element_type=jnp.float32)

        for nb in (1, 2, 3, 4):
            @pl.when(c // 2 + 1 == nb)
            def _(nb=nb):
                kv_len = nb * 512
                rowi = row0 + lax.broadcasted_iota(jnp.int32, (CHUNK, kv_len), 0)
                coli = lax.broadcasted_iota(jnp.int32, (CHUNK, kv_len), 1)
                bias = jnp.where((coli // BLK) <= (rowi // BLK),
                                 0.0, -1e9).astype(jnp.float32)

                ctxs = []
                for h in range(HEADS):
                    q = q_all[:, h * DH:(h + 1) * DH]
                    k = k_ref[h, 0:kv_len, :]
                    s = lax.dot_general(q, k, (((1,), (1,)), ((), ())),
                                        preferred_element_type=jnp.float32)
                    s = s * SCALE + bias
                    m = jnp.max(s, axis=1, keepdims=True)
                    p = jnp.exp(s - m)
                    l = jnp.sum(p, axis=1, keepdims=True)
                    ctxs.append(jnp.dot(p, v_ref[h, 0:kv_len, :],
                                        preferred_element_type=jnp.float32) / l)

                ctx_all = jnp.concatenate(ctxs, axis=1)
                out_ref[pl.ds(row0, CHUNK), :] = jnp.dot(
                    ctx_all, wo_ref[:, :], preferred_element_type=jnp.float32)

    def mk_l(s):
        return pltpu.make_async_remote_copy(
            src_ref=comm_l.at[s],
            dst_ref=comm_l.at[s + 1],
            send_sem=rsl_send.at[s],
            recv_sem=rsl_recv.at[s],
            device_id=(left,),
            device_id_type=pl.DeviceIdType.MESH,
        )

    def mk_r(s):
        return pltpu.make_async_remote_copy(
            src_ref=comm_r.at[s],
            dst_ref=comm_r.at[s + 1],
            send_sem=rsr_send.at[s],
            recv_sem=rsr_recv.at[s],
            device_id=(right,),
            device_id_type=pl.DeviceIdType.MESH,
        )

    def add_l(s, c):
        comm_l[s + 1, :, :] = comm_l[s + 1, :, :] + out_ref[
            pl.ds(c * CHUNK, CHUNK), 0:HALF]

    def add_r(s, c):
        comm_r[s + 1, :, :] = comm_r[s + 1, :, :] + out_ref[
            pl.ds(c * CHUNK, CHUNK), HALF:DM]

    compute_chunk(my)
    comm_l[0, :, :] = out_ref[pl.ds(my * CHUNK, CHUNK), 0:HALF]
    comm_r[0, :, :] = out_ref[pl.ds(my * CHUNK, CHUNK), HALF:DM]
    mk_l(0).start()
    mk_r(0).start()

    def pos_body(t, _):
        k = (t + 1) // 2
        off = jnp.where(t % 2 == 1, k, -k)
        cc = lax.rem(my + off + N_DEV, N_DEV)
        compute_chunk(cc)

        @pl.when(t % 2 == 1)
        def _():
            s = (t - 1) // 2
            mk_l(s).wait()
            add_l(s, cc)
            mk_l(s + 1).start()

        @pl.when(t % 2 == 0)
        def _():
            s = t // 2 - 1
            mk_r(s).wait()
            add_r(s, cc)
            mk_r(s + 1).start()

        return 0

    lax.fori_loop(1, N_DEV, pos_body, 0)

    mk_r(3).wait()
    add_r(3, lax.rem(my + N_DEV - 4, N_DEV))
    mk_r(4).start()
    mk_l(4).wait()
    add_l(4, lax.rem(my + 5, N_DEV))
    mk_l(5).start()
    mk_r(4).wait()
    add_r(4, lax.rem(my + N_DEV - 5, N_DEV))
    mk_r(5).start()
    mk_l(5).wait()
    add_l(5, lax.rem(my + 6, N_DEV))
    mk_l(6).start()
    mk_r(5).wait()
    add_r(5, lax.rem(my + N_DEV - 6, N_DEV))
    mk_r(6).start()
    mk_l(6).wait()
    add_l(6, lax.rem(my + 7, N_DEV))
    mk_r(6).wait()
    add_r(6, lax.rem(my + N_DEV - 7, N_DEV))

    out_ref[pl.ds(left * CHUNK, CHUNK), 0:HALF] = comm_l[N_DEV - 1, :, :]
    out_ref[pl.ds(right * CHUNK, CHUNK), HALF:DM] = comm_r[N_DEV - 1, :, :]
    for h in range(N_DEV - 1):
        send_slot = (N_DEV - 1 + h) % N_DEV
        r_rdma = pltpu.make_async_remote_copy(
            src_ref=comm_l.at[send_slot],
            dst_ref=comm_l.at[h],
            send_sem=agr_send.at[h],
            recv_sem=agr_recv.at[h],
            device_id=(right,),
            device_id_type=pl.DeviceIdType.MESH,
        )
        l_rdma = pltpu.make_async_remote_copy(
            src_ref=comm_r.at[send_slot],
            dst_ref=comm_r.at[h],
            send_sem=agl_send.at[h],
            recv_sem=agl_recv.at[h],
            device_id=(left,),
            device_id_type=pl.DeviceIdType.MESH,
        )
        r_rdma.start()
        l_rdma.start()
        r_rdma.wait()
        l_rdma.wait()
        cr = lax.rem(my + 2 * N_DEV - 2 - h, N_DEV)
        cl = lax.rem(my + 2 + h, N_DEV)
        out_ref[pl.ds(cr * CHUNK, CHUNK), 0:HALF] = comm_l[h, :, :]
        out_ref[pl.ds(cl * CHUNK, CHUNK), HALF:DM] = comm_r[h, :, :]

    @functools.partial(pl.run_scoped, sem=pltpu.SemaphoreType.REGULAR)
    def _(sem):
        for nbr in (left, right):
            pl.semaphore_signal(sem, inc=1, device_id=(nbr,),
                                device_id_type=pl.DeviceIdType.MESH)
        pl.semaphore_wait(sem, 2)


def kernel(x, Wq, K_ext, V_ext, Wo):
    i = lax.axis_index("i")
    x2 = x[0]
    k_loc = lax.dynamic_slice_in_dim(K_ext[0], i * HEADS, HEADS, axis=1)
    v_loc = lax.dynamic_slice_in_dim(V_ext[0], i * HEADS, HEADS, axis=1)
    k_loc = jnp.transpose(k_loc, (1, 0, 2))
    v_loc = jnp.transpose(v_loc, (1, 0, 2))

    out = pl.pallas_call(
        _body,
        out_shape=jax.ShapeDtypeStruct((SQ, DM), jnp.float32),
        in_specs=[pl.BlockSpec(memory_space=pltpu.VMEM)] * 5,
        out_specs=pl.BlockSpec(memory_space=pltpu.VMEM),
        scratch_shapes=[
            pltpu.VMEM((N_DEV, CHUNK, HALF), jnp.float32),
            pltpu.VMEM((N_DEV, CHUNK, HALF), jnp.float32),
            pltpu.SemaphoreType.DMA((N_DEV - 1,)),
            pltpu.SemaphoreType.DMA((N_DEV - 1,)),
            pltpu.SemaphoreType.DMA((N_DEV - 1,)),
            pltpu.SemaphoreType.DMA((N_DEV - 1,)),
            pltpu.SemaphoreType.DMA((N_DEV - 1,)),
            pltpu.SemaphoreType.DMA((N_DEV - 1,)),
            pltpu.SemaphoreType.DMA((N_DEV - 1,)),
            pltpu.SemaphoreType.DMA((N_DEV - 1,)),
        ],
        compiler_params=pltpu.CompilerParams(
            collective_id=0,
            vmem_limit_bytes=64 * 1024 * 1024,
        ),
    )(x2, Wq, k_loc, v_loc, Wo)
    return out[None]


# device time: 190617 ns/iter; 1.7880x vs baseline; 1.0012x over previous
import functools

import jax
import jax.numpy as jnp
from jax import lax
from jax.experimental import pallas as pl
from jax.experimental.pallas import tpu as pltpu

N_DEV = 8
HEADS = 8
DH = 128
SQ = 2048
DM = 1024
BLK = 64
SCALE = 0.08838834764831843
CHUNK = SQ // N_DEV
HALF = DM // 2


def _body(x_ref, wq_ref, k_ref, v_ref, wo_ref, out_ref,
          comm_l, comm_r,
          rsl_send, rsl_recv, rsr_send, rsr_recv,
          agr_send, agr_recv, agl_send, agl_recv):
    my = lax.axis_index("i")
    left = lax.rem(my + N_DEV - 1, N_DEV)
    right = lax.rem(my + 1, N_DEV)

    barrier = pltpu.get_barrier_semaphore()
    for nbr in (left, right):
        pl.semaphore_signal(barrier, inc=1, device_id=(nbr,),
                            device_id_type=pl.DeviceIdType.MESH)
    pl.semaphore_wait(barrier, 2)

    def compute_chunk(c):
        row0 = c * CHUNK
        xc = x_ref[pl.ds(row0, CHUNK), :]
        q_all = jnp.dot(xc, wq_ref[:, :],
                        preferred_element_type=jnp.float32)

        for nb in (1, 2, 3, 4):
            @pl.when(c // 2 + 1 == nb)
            def _(nb=nb):
                kv_len = nb * 512
                rowi = row0 + lax.broadcasted_iota(jnp.int32, (CHUNK, kv_len), 0)
                coli = lax.broadcasted_iota(jnp.int32, (CHUNK, kv_len), 1)
                bias = jnp.where((coli // BLK) <= (rowi // BLK),
                                 0.0, -1e9).astype(jnp.float32)

                ctxs = []
                for h in range(HEADS):
                    q = q_all[:, h * DH:(h + 1) * DH]
                    k = k_ref[h, 0:kv_len, :]
                    s = lax.dot_general(q, k, (((1,), (1,)), ((), ())),
                                        preferred_element_type=jnp.float32)
                    s = s * SCALE + bias
                    m = jnp.max(s, axis=1, keepdims=True)
                    p = jnp.exp(s - m)
                    l = jnp.sum(p, axis=1, keepdims=True)
                    ctxs.append(jnp.dot(p, v_ref[h, 0:kv_len, :],
                                        preferred_element_type=jnp.float32) / l)

                ctx_all = jnp.concatenate(ctxs, axis=1)
                out_ref[pl.ds(row0, CHUNK), :] = jnp.dot(
                    ctx_all, wo_ref[:, :], preferred_element_type=jnp.float32)

    def mk_l(s):
        return pltpu.make_async_remote_copy(
            src_ref=comm_l.at[s],
            dst_ref=comm_l.at[s + 1],
            send_sem=rsl_send.at[s],
            recv_sem=rsl_recv.at[s],
            device_id=(left,),
            device_id_type=pl.DeviceIdType.MESH,
        )

    def mk_r(s):
        return pltpu.make_async_remote_copy(
            src_ref=comm_r.at[s],
            dst_ref=comm_r.at[s + 1],
            send_sem=rsr_send.at[s],
            recv_sem=rsr_recv.at[s],
            device_id=(right,),
            device_id_type=pl.DeviceIdType.MESH,
        )

    def add_l(s, c):
        comm_l[s + 1, :, :] = comm_l[s + 1, :, :] + out_ref[
            pl.ds(c * CHUNK, CHUNK), 0:HALF]

    def add_r(s, c):
        comm_r[s + 1, :, :] = comm_r[s + 1, :, :] + out_ref[
            pl.ds(c * CHUNK, CHUNK), HALF:DM]

    compute_chunk(my)
    comm_l[0, :, :] = out_ref[pl.ds(my * CHUNK, CHUNK), 0:HALF]
    comm_r[0, :, :] = out_ref[pl.ds(my * CHUNK, CHUNK), HALF:DM]
    mk_l(0).start()
    mk_r(0).start()

    def pos_body(t, _):
        k = (t + 1) // 2
        off = jnp.where(t % 2 == 1, k, -k)
        cc = lax.rem(my + off + N_DEV, N_DEV)
        compute_chunk(cc)

        @pl.when(t % 2 == 1)
        def _():
            s = (t - 1) // 2
            mk_l(s).wait()
            add_l(s, cc)
            mk_l(s + 1).start()

        @pl.when(t % 2 == 0)
        def _():
            s = t // 2 - 1
            mk_r(s).wait()
            add_r(s, cc)
            mk_r(s + 1).start()

        return 0

    lax.fori_loop(1, N_DEV, pos_body, 0)

    mk_r(3).wait()
    add_r(3, lax.rem(my + N_DEV - 4, N_DEV))
    mk_r(4).start()
    mk_l(4).wait()
    add_l(4, lax.rem(my + 5, N_DEV))
    mk_l(5).start()
    mk_r(4).wait()
    add_r(4, lax.rem(my + N_DEV - 5, N_DEV))
    mk_r(5).start()
    mk_l(5).wait()
    add_l(5, lax.rem(my + 6, N_DEV))
    mk_l(6).start()
    mk_r(5).wait()
    add_r(5, lax.rem(my + N_DEV - 6, N_DEV))
    mk_r(6).start()
    def mk_ag_r(h):
        return pltpu.make_async_remote_copy(
            src_ref=comm_l.at[(N_DEV - 1 + h) % N_DEV],
            dst_ref=comm_l.at[h],
            send_sem=agr_send.at[h],
            recv_sem=agr_recv.at[h],
            device_id=(right,),
            device_id_type=pl.DeviceIdType.MESH,
        )

    def mk_ag_l(h):
        return pltpu.make_async_remote_copy(
            src_ref=comm_r.at[(N_DEV - 1 + h) % N_DEV],
            dst_ref=comm_r.at[h],
            send_sem=agl_send.at[h],
            recv_sem=agl_recv.at[h],
            device_id=(left,),
            device_id_type=pl.DeviceIdType.MESH,
        )

    mk_l(6).wait()
    add_l(6, lax.rem(my + 7, N_DEV))
    mk_ag_r(0).start()
    mk_r(6).wait()
    add_r(6, lax.rem(my + N_DEV - 7, N_DEV))
    mk_ag_l(0).start()
    out_ref[pl.ds(left * CHUNK, CHUNK), 0:HALF] = comm_l[N_DEV - 1, :, :]
    out_ref[pl.ds(right * CHUNK, CHUNK), HALF:DM] = comm_r[N_DEV - 1, :, :]
    for h in range(N_DEV - 1):
        mk_ag_r(h).wait()
        mk_ag_l(h).wait()
        if h + 1 < N_DEV - 1:
            mk_ag_r(h + 1).start()
            mk_ag_l(h + 1).start()
        cr = lax.rem(my + 2 * N_DEV - 2 - h, N_DEV)
        cl = lax.rem(my + 2 + h, N_DEV)
        out_ref[pl.ds(cr * CHUNK, CHUNK), 0:HALF] = comm_l[h, :, :]
        out_ref[pl.ds(cl * CHUNK, CHUNK), HALF:DM] = comm_r[h, :, :]

    @functools.partial(pl.run_scoped, sem=pltpu.SemaphoreType.REGULAR)
    def _(sem):
        for nbr in (left, right):
            pl.semaphore_signal(sem, inc=1, device_id=(nbr,),
                                device_id_type=pl.DeviceIdType.MESH)
        pl.semaphore_wait(sem, 2)


def kernel(x, Wq, K_ext, V_ext, Wo):
    i = lax.axis_index("i")
    x2 = x[0]
    k_loc = lax.dynamic_slice_in_dim(K_ext[0], i * HEADS, HEADS, axis=1)
    v_loc = lax.dynamic_slice_in_dim(V_ext[0], i * HEADS, HEADS, axis=1)
    k_loc = jnp.transpose(k_loc, (1, 0, 2))
    v_loc = jnp.transpose(v_loc, (1, 0, 2))

    out = pl.pallas_call(
        _body,
        out_shape=jax.ShapeDtypeStruct((SQ, DM), jnp.float32),
        in_specs=[pl.BlockSpec(memory_space=pltpu.VMEM)] * 5,
        out_specs=pl.BlockSpec(memory_space=pltpu.VMEM),
        scratch_shapes=[
            pltpu.VMEM((N_DEV, CHUNK, HALF), jnp.float32),
            pltpu.VMEM((N_DEV, CHUNK, HALF), jnp.float32),
            pltpu.SemaphoreType.DMA((N_DEV - 1,)),
            pltpu.SemaphoreType.DMA((N_DEV - 1,)),
            pltpu.SemaphoreType.DMA((N_DEV - 1,)),
            pltpu.SemaphoreType.DMA((N_DEV - 1,)),
            pltpu.SemaphoreType.DMA((N_DEV - 1,)),
            pltpu.SemaphoreType.DMA((N_DEV - 1,)),
            pltpu.SemaphoreType.DMA((N_DEV - 1,)),
            pltpu.SemaphoreType.DMA((N_DEV - 1,)),
        ],
        compiler_params=pltpu.CompilerParams(
            collective_id=0,
            vmem_limit_bytes=64 * 1024 * 1024,
        ),
    )(x2, Wq, k_loc, v_loc, Wo)
    return out[None]
